# baseline (speedup 1.0000x reference)
import functools

import jax
import jax.numpy as jnp
from jax import lax
from jax.experimental import pallas as pl
from jax.experimental.pallas import tpu as pltpu
from jax.experimental.pallas import tpu_sc as plsc

N = 10000
E = 320000
D = 128

NC = 2
NS = 16
NW = NC * NS
K = 128
H = K // 2
CHUNKS = 80
EPT = CHUNKS * K
E_PAD = NW * EPT
NPAD = 10112
ZROWS = NPAD // NS
DUMP = NPAD - N
IB = 16
NB = CHUNKS // IB

_MESH = plsc.VectorSubcoreMesh(core_axis_name="c", subcore_axis_name="s")


def _vcopy_idx(src_v, i, dst_v):
    base = pl.multiple_of(i * K, K)
    for k in range(K // 16):
        dst_v[pl.ds(k * 16, 16)] = src_v[pl.ds(base + k * 16, 16)]


@functools.partial(
    pl.kernel, mesh=_MESH,
    out_type=jax.ShapeDtypeStruct((NW, ZROWS, D), jnp.float32),
    scratch_types=[
        pltpu.VMEM((IB * K,), jnp.int32),
        pltpu.VMEM((IB * K,), jnp.int32),
        pltpu.VMEM((K,), jnp.int32),
        pltpu.VMEM((K,), jnp.int32),
        pltpu.VMEM((K, D), jnp.float32),
        pltpu.VMEM((K, D), jnp.float32),
        pltpu.VMEM_SHARED((NPAD, D), jnp.float32),
        pltpu.SemaphoreType.DMA,
        pltpu.SemaphoreType.DMA,
        pltpu.SemaphoreType.DMA,
        pltpu.SemaphoreType.DMA,
    ],
)
def _seg(table_hbm, src_hbm, dst_hbm, zeros_hbm, out_hbm,
         srcs_v, dsts_v, dstc0, dstc1, rows0, rows1, acc,
         sem0, sem1, ssem0, ssem1):
    c = lax.axis_index("c")
    s = lax.axis_index("s")
    wid = c * NS + s
    pltpu.sync_copy(zeros_hbm.at[pl.ds(s * ZROWS, ZROWS)],
                    acc.at[pl.ds(s * ZROWS, ZROWS)])
    plsc.subcore_barrier()

    Q = K // 4

    def fire(l, rows, sem):
        base = pl.multiple_of(l * K, K)
        for q in range(4):
            pltpu.async_copy(table_hbm.at[srcs_v.at[pl.ds(base + q * Q, Q)]],
                             rows.at[pl.ds(q * Q, Q)], sem)

    def wait(rows, sem):
        pltpu.make_async_copy(table_hbm.at[srcs_v.at[pl.ds(0, K)]],
                              rows, sem).wait()

    def wait_scatter(rows, dstc, ssem):
        pltpu.make_async_copy(rows, acc.at[dstc], ssem).wait()

    def outer(b, carry):
        blk = wid * EPT + b * (IB * K)
        pltpu.sync_copy(src_hbm.at[pl.ds(blk, IB * K)], srcs_v)
        pltpu.sync_copy(dst_hbm.at[pl.ds(blk, IB * K)], dsts_v)
        fire(0, rows0, sem0)
        fire(1, rows1, sem1)

        def body(i, carry2):
            l0 = 2 * i
            l1 = l0 + 1
            wait(rows0, sem0)
            _vcopy_idx(dsts_v, l0, dstc0)
            pltpu.async_copy(rows0, acc.at[dstc0], ssem0, add=True)
            wait(rows1, sem1)
            _vcopy_idx(dsts_v, l1, dstc1)
            pltpu.async_copy(rows1, acc.at[dstc1], ssem1, add=True)
            wait_scatter(rows0, dstc0, ssem0)

            @pl.when(i < IB // 2 - 1)
            def _():
                fire(l0 + 2, rows0, sem0)

            wait_scatter(rows1, dstc1, ssem1)

            @pl.when(i < IB // 2 - 1)
            def _():
                fire(l1 + 2, rows1, sem1)

            return carry2

        lax.fori_loop(0, IB // 2, body, 0)
        return carry

    lax.fori_loop(0, NB, outer, 0)
    plsc.subcore_barrier()
    pltpu.sync_copy(acc.at[pl.ds(s * ZROWS, ZROWS)], out_hbm.at[wid])


@functools.partial(
    pl.kernel, mesh=_MESH,
    out_type=jax.ShapeDtypeStruct((NW, ZROWS, D), jnp.float32),
    scratch_types=[
        pltpu.VMEM((K,), jnp.int32),
        pltpu.VMEM((K, D), jnp.float32),
        pltpu.VMEM_SHARED((NPAD, D), jnp.float32),
    ],
)
def _deg(ones_hbm, dst_hbm, zeros_hbm, out_hbm, dst_v, ones_v, acc):
    c = lax.axis_index("c")
    s = lax.axis_index("s")
    wid = c * NS + s
    pltpu.sync_copy(zeros_hbm.at[pl.ds(s * ZROWS, ZROWS)],
                    acc.at[pl.ds(s * ZROWS, ZROWS)])
    pltpu.sync_copy(ones_hbm, ones_v)
    plsc.subcore_barrier()

    def body(i, carry):
        base = wid * EPT + i * K
        pltpu.sync_copy(dst_hbm.at[pl.ds(base, K)], dst_v)
        pltpu.sync_copy(ones_v, acc.at[dst_v], add=True)
        return carry

    lax.fori_loop(0, CHUNKS, body, 0)
    plsc.subcore_barrier()
    pltpu.sync_copy(acc.at[pl.ds(s * ZROWS, ZROWS)], out_hbm.at[wid])


def _layer0_body(x_ref, a_ref, b_ref, da_ref, db_ref, ws_ref, wn_ref,
                 h_ref, inv_ref):
    inv = 1.0 / jnp.maximum(da_ref[...] + db_ref[...], 1.0)
    mean = (a_ref[...] + b_ref[...]) * inv
    h = (jnp.dot(x_ref[...], ws_ref[...], preferred_element_type=jnp.float32)
         + jnp.dot(mean, wn_ref[...], preferred_element_type=jnp.float32))
    h_ref[...] = jnp.maximum(h, 0.0)
    inv_ref[...] = inv


def _layerN_body(h_ref, a_ref, b_ref, inv_ref, ws_ref, wn_ref, o_ref, *, relu):
    mean = (a_ref[...] + b_ref[...]) * inv_ref[...]
    o = (jnp.dot(h_ref[...], ws_ref[...], preferred_element_type=jnp.float32)
         + jnp.dot(mean, wn_ref[...], preferred_element_type=jnp.float32))
    if relu:
        o = jnp.maximum(o, 0.0)
    o_ref[...] = o


_BLK = ZROWS
_GRID = NPAD // _BLK


def _row_spec(w):
    return pl.BlockSpec((_BLK, w), lambda i: (i, 0))


def _full_spec(r, w):
    return pl.BlockSpec((r, w), lambda i: (0, 0))


def _tc_layer0(x, accA, accB, degA, degB, ws, wn):
    return pl.pallas_call(
        _layer0_body,
        grid=(_GRID,),
        in_specs=[_row_spec(128), _row_spec(128), _row_spec(128),
                  _row_spec(128), _row_spec(128),
                  _full_spec(128, 128), _full_spec(128, 128)],
        out_specs=[_row_spec(128), _row_spec(128)],
        out_shape=[jax.ShapeDtypeStruct((NPAD, 128), jnp.float32),
                   jax.ShapeDtypeStruct((NPAD, 128), jnp.float32)],
    )(x, accA, accB, degA, degB, ws, wn)


def _tc_layerN(h, accA, accB, invb, ws, wn, relu):
    return pl.pallas_call(
        functools.partial(_layerN_body, relu=relu),
        grid=(_GRID,),
        in_specs=[_row_spec(128), _row_spec(128), _row_spec(128),
                  _row_spec(128), _full_spec(128, 128), _full_spec(128, 128)],
        out_specs=_row_spec(128),
        out_shape=jax.ShapeDtypeStruct((NPAD, 128), jnp.float32),
    )(h, accA, accB, invb, ws, wn)


def kernel(x, edge_index, W_self0, W_neigh0, W_self1, W_neigh1, W_self2, W_neigh2):
    src = edge_index[0]
    dst = edge_index[1]
    pad = E_PAD - E
    pad_src = (jnp.arange(pad, dtype=jnp.int32) * 97) % N
    pad_dst = N + (jnp.arange(pad, dtype=jnp.int32) % DUMP)
    srcp = jnp.concatenate([src, pad_src])
    dstp = jnp.concatenate([dst, pad_dst])
    zeros = jnp.zeros((NPAD, D), jnp.float32)
    ones = jnp.ones((K, D), jnp.float32)
    xp = jnp.pad(x, ((0, DUMP), (0, 0)))

    degr = _deg(ones, dstp, zeros).reshape(NC, NPAD, D)

    acc0 = _seg(xp, srcp, dstp, zeros).reshape(NC, NPAD, D)
    h1, invb = _tc_layer0(xp, acc0[0], acc0[1], degr[0], degr[1],
                          W_self0, W_neigh0)

    acc1 = _seg(h1, srcp, dstp, zeros).reshape(NC, NPAD, D)
    h2 = _tc_layerN(h1, acc1[0], acc1[1], invb, W_self1, W_neigh1, relu=True)

    acc2 = _seg(h2, srcp, dstp, zeros).reshape(NC, NPAD, D)
    ws2 = jnp.pad(W_self2, ((0, 0), (0, 128 - 40)))
    wn2 = jnp.pad(W_neigh2, ((0, 0), (0, 128 - 40)))
    h3 = _tc_layerN(h2, acc2[0], acc2[1], invb, ws2, wn2, relu=False)
    return h3[:N, :40]


# speedup vs baseline: 1.0016x; 1.0016x over previous
import functools

import jax
import jax.numpy as jnp
from jax import lax
from jax.experimental import pallas as pl
from jax.experimental.pallas import tpu as pltpu
from jax.experimental.pallas import tpu_sc as plsc

N = 10000
E = 320000
D = 128

NC = 2
NS = 16
NW = NC * NS
K = 128
H = K // 2
CHUNKS = 80
EPT = CHUNKS * K
E_PAD = NW * EPT
NPAD = 10112
ZROWS = NPAD // NS
DUMP = NPAD - N
IB = 16
NB = CHUNKS // IB

_MESH = plsc.VectorSubcoreMesh(core_axis_name="c", subcore_axis_name="s")


def _vcopy_idx(src_v, i, dst_v):
    base = pl.multiple_of(i * K, K)
    for k in range(K // 16):
        dst_v[pl.ds(k * 16, 16)] = src_v[pl.ds(base + k * 16, 16)]


@functools.partial(
    pl.kernel, mesh=_MESH,
    out_type=jax.ShapeDtypeStruct((NW, ZROWS, D), jnp.float32),
    scratch_types=[
        pltpu.VMEM((IB * K,), jnp.int32),
        pltpu.VMEM((IB * K,), jnp.int32),
        pltpu.VMEM((K,), jnp.int32),
        pltpu.VMEM((K,), jnp.int32),
        pltpu.VMEM((K, D), jnp.float32),
        pltpu.VMEM((K, D), jnp.float32),
        pltpu.VMEM_SHARED((NPAD, D), jnp.float32),
        pltpu.SemaphoreType.DMA,
        pltpu.SemaphoreType.DMA,
        pltpu.SemaphoreType.DMA,
        pltpu.SemaphoreType.DMA,
    ],
)
def _seg(table_hbm, src_hbm, dst_hbm, zeros_hbm, out_hbm,
         srcs_v, dsts_v, dstc0, dstc1, rows0, rows1, acc,
         sem0, sem1, ssem0, ssem1):
    c = lax.axis_index("c")
    s = lax.axis_index("s")
    wid = c * NS + s
    pltpu.sync_copy(zeros_hbm.at[pl.ds(s * ZROWS, ZROWS)],
                    acc.at[pl.ds(s * ZROWS, ZROWS)])
    plsc.subcore_barrier()

    def fire(l, rows, sem):
        base = pl.multiple_of(l * K, K)
        for q in range(2):
            pltpu.async_copy(table_hbm.at[srcs_v.at[pl.ds(base + q * H, H)]],
                             rows.at[pl.ds(q * H, H)], sem)

    def wait(rows, sem):
        pltpu.make_async_copy(table_hbm.at[srcs_v.at[pl.ds(0, K)]],
                              rows, sem).wait()

    def wait_scatter(rows, dstc, ssem):
        pltpu.make_async_copy(rows, acc.at[dstc], ssem).wait()

    def outer(b, carry):
        blk = wid * EPT + b * (IB * K)
        pltpu.sync_copy(src_hbm.at[pl.ds(blk, IB * K)], srcs_v)
        pltpu.sync_copy(dst_hbm.at[pl.ds(blk, IB * K)], dsts_v)
        fire(0, rows0, sem0)
        fire(1, rows1, sem1)

        def body(i, carry2):
            l0 = 2 * i
            l1 = l0 + 1
            wait(rows0, sem0)
            _vcopy_idx(dsts_v, l0, dstc0)
            pltpu.async_copy(rows0, acc.at[dstc0], ssem0, add=True)
            wait(rows1, sem1)
            _vcopy_idx(dsts_v, l1, dstc1)
            pltpu.async_copy(rows1, acc.at[dstc1], ssem1, add=True)
            wait_scatter(rows0, dstc0, ssem0)

            @pl.when(i < IB // 2 - 1)
            def _():
                fire(l0 + 2, rows0, sem0)

            wait_scatter(rows1, dstc1, ssem1)

            @pl.when(i < IB // 2 - 1)
            def _():
                fire(l1 + 2, rows1, sem1)

            return carry2

        lax.fori_loop(0, IB // 2, body, 0)
        return carry

    lax.fori_loop(0, NB, outer, 0)
    plsc.subcore_barrier()
    pltpu.sync_copy(acc.at[pl.ds(s * ZROWS, ZROWS)], out_hbm.at[wid])


@functools.partial(
    pl.kernel, mesh=_MESH,
    out_type=jax.ShapeDtypeStruct((NW, ZROWS, D), jnp.float32),
    scratch_types=[
        pltpu.VMEM((K,), jnp.int32),
        pltpu.VMEM((K, D), jnp.float32),
        pltpu.VMEM_SHARED((NPAD, D), jnp.float32),
    ],
)
def _deg(ones_hbm, dst_hbm, zeros_hbm, out_hbm, dst_v, ones_v, acc):
    c = lax.axis_index("c")
    s = lax.axis_index("s")
    wid = c * NS + s
    pltpu.sync_copy(zeros_hbm.at[pl.ds(s * ZROWS, ZROWS)],
                    acc.at[pl.ds(s * ZROWS, ZROWS)])
    pltpu.sync_copy(ones_hbm, ones_v)
    plsc.subcore_barrier()

    def body(i, carry):
        base = wid * EPT + i * K
        pltpu.sync_copy(dst_hbm.at[pl.ds(base, K)], dst_v)
        pltpu.sync_copy(ones_v, acc.at[dst_v], add=True)
        return carry

    lax.fori_loop(0, CHUNKS, body, 0)
    plsc.subcore_barrier()
    pltpu.sync_copy(acc.at[pl.ds(s * ZROWS, ZROWS)], out_hbm.at[wid])


def _layer0_body(x_ref, a_ref, b_ref, da_ref, db_ref, ws_ref, wn_ref,
                 h_ref, inv_ref):
    inv = 1.0 / jnp.maximum(da_ref[...] + db_ref[...], 1.0)
    mean = (a_ref[...] + b_ref[...]) * inv
    h = (jnp.dot(x_ref[...], ws_ref[...], preferred_element_type=jnp.float32)
         + jnp.dot(mean, wn_ref[...], preferred_element_type=jnp.float32))
    h_ref[...] = jnp.maximum(h, 0.0)
    inv_ref[...] = inv


def _layerN_body(h_ref, a_ref, b_ref, inv_ref, ws_ref, wn_ref, o_ref, *, relu):
    mean = (a_ref[...] + b_ref[...]) * inv_ref[...]
    o = (jnp.dot(h_ref[...], ws_ref[...], preferred_element_type=jnp.float32)
         + jnp.dot(mean, wn_ref[...], preferred_element_type=jnp.float32))
    if relu:
        o = jnp.maximum(o, 0.0)
    o_ref[...] = o


_BLK = ZROWS
_GRID = NPAD // _BLK


def _row_spec(w):
    return pl.BlockSpec((_BLK, w), lambda i: (i, 0))


def _full_spec(r, w):
    return pl.BlockSpec((r, w), lambda i: (0, 0))


def _tc_layer0(x, accA, accB, degA, degB, ws, wn):
    return pl.pallas_call(
        _layer0_body,
        grid=(_GRID,),
        in_specs=[_row_spec(128), _row_spec(128), _row_spec(128),
                  _row_spec(128), _row_spec(128),
                  _full_spec(128, 128), _full_spec(128, 128)],
        out_specs=[_row_spec(128), _row_spec(128)],
        out_shape=[jax.ShapeDtypeStruct((NPAD, 128), jnp.float32),
                   jax.ShapeDtypeStruct((NPAD, 128), jnp.float32)],
    )(x, accA, accB, degA, degB, ws, wn)


def _tc_layerN(h, accA, accB, invb, ws, wn, relu):
    return pl.pallas_call(
        functools.partial(_layerN_body, relu=relu),
        grid=(_GRID,),
        in_specs=[_row_spec(128), _row_spec(128), _row_spec(128),
                  _row_spec(128), _full_spec(128, 128), _full_spec(128, 128)],
        out_specs=_row_spec(128),
        out_shape=jax.ShapeDtypeStruct((NPAD, 128), jnp.float32),
    )(h, accA, accB, invb, ws, wn)


def kernel(x, edge_index, W_self0, W_neigh0, W_self1, W_neigh1, W_self2, W_neigh2):
    src = edge_index[0]
    dst = edge_index[1]
    pad = E_PAD - E
    pad_src = (jnp.arange(pad, dtype=jnp.int32) * 97) % N
    pad_dst = N + (jnp.arange(pad, dtype=jnp.int32) % DUMP)
    srcp = jnp.concatenate([src, pad_src])
    dstp = jnp.concatenate([dst, pad_dst])
    zeros = jnp.zeros((NPAD, D), jnp.float32)
    ones = jnp.ones((K, D), jnp.float32)
    xp = jnp.pad(x, ((0, DUMP), (0, 0)))

    degr = _deg(ones, dstp, zeros).reshape(NC, NPAD, D)

    acc0 = _seg(xp, srcp, dstp, zeros).reshape(NC, NPAD, D)
    h1, invb = _tc_layer0(xp, acc0[0], acc0[1], degr[0], degr[1],
                          W_self0, W_neigh0)

    acc1 = _seg(h1, srcp, dstp, zeros).reshape(NC, NPAD, D)
    h2 = _tc_layerN(h1, acc1[0], acc1[1], invb, W_self1, W_neigh1, relu=True)

    acc2 = _seg(h2, srcp, dstp, zeros).reshape(NC, NPAD, D)
    ws2 = jnp.pad(W_self2, ((0, 0), (0, 128 - 40)))
    wn2 = jnp.pad(W_neigh2, ((0, 0), (0, 128 - 40)))
    h3 = _tc_layerN(h2, acc2[0], acc2[1], invb, ws2, wn2, relu=False)
    return h3[:N, :40]


# speedup vs baseline: 1.0700x; 1.0683x over previous
import functools

import jax
import jax.numpy as jnp
from jax import lax
from jax.experimental import pallas as pl
from jax.experimental.pallas import tpu as pltpu
from jax.experimental.pallas import tpu_sc as plsc

N = 10000
E = 320000
D = 128

NC = 2
NS = 16
NW = NC * NS
K = 128
H = K // 2
CHUNKS = 80
EPT = CHUNKS * K
E_PAD = NW * EPT
NPAD = 10112
ZROWS = NPAD // NS
DUMP = NPAD - N
IB = 16
NB = CHUNKS // IB

_MESH = plsc.VectorSubcoreMesh(core_axis_name="c", subcore_axis_name="s")


def _vcopy_idx(src_v, i, dst_v):
    base = pl.multiple_of(i * K, K)
    for k in range(K // 16):
        dst_v[pl.ds(k * 16, 16)] = src_v[pl.ds(base + k * 16, 16)]


@functools.partial(
    pl.kernel, mesh=_MESH,
    out_type=jax.ShapeDtypeStruct((NW, ZROWS, D), jnp.float32),
    scratch_types=[
        pltpu.VMEM((IB * K,), jnp.int32),
        pltpu.VMEM((IB * K,), jnp.int32),
        pltpu.VMEM((K,), jnp.int32),
        pltpu.VMEM((K,), jnp.int32),
        pltpu.VMEM((K, D), jnp.float32),
        pltpu.VMEM((K, D), jnp.float32),
        pltpu.VMEM_SHARED((NPAD, D), jnp.float32),
        pltpu.SemaphoreType.DMA,
        pltpu.SemaphoreType.DMA,
    ],
)
def _seg(table_hbm, src_hbm, dst_hbm, zeros_hbm, out_hbm,
         srcs_v, dsts_v, dstc0, dstc1, rows0, rows1, acc, sem0, sem1):
    c = lax.axis_index("c")
    s = lax.axis_index("s")
    wid = c * NS + s
    pltpu.sync_copy(zeros_hbm.at[pl.ds(s * ZROWS, ZROWS)],
                    acc.at[pl.ds(s * ZROWS, ZROWS)])
    plsc.subcore_barrier()

    def fire(l, rows, sem):
        base = pl.multiple_of(l * K, K)
        for q in range(2):
            pltpu.async_copy(table_hbm.at[srcs_v.at[pl.ds(base + q * H, H)]],
                             rows.at[pl.ds(q * H, H)], sem)

    def wait(rows, sem):
        pltpu.make_async_copy(table_hbm.at[srcs_v.at[pl.ds(0, K)]],
                              rows, sem).wait()

    def outer(b, carry):
        blk = wid * EPT + b * (IB * K)
        pltpu.sync_copy(src_hbm.at[pl.ds(blk, IB * K)], srcs_v)
        pltpu.sync_copy(dst_hbm.at[pl.ds(blk, IB * K)], dsts_v)
        fire(0, rows0, sem0)

        def body(i, carry2):
            l0 = 2 * i
            l1 = l0 + 1
            wait(rows0, sem0)
            fire(l1, rows1, sem1)
            _vcopy_idx(dsts_v, l0, dstc0)
            pltpu.sync_copy(rows0, acc.at[dstc0], add=True)
            wait(rows1, sem1)

            @pl.when(i < IB // 2 - 1)
            def _():
                fire(l0 + 2, rows0, sem0)

            _vcopy_idx(dsts_v, l1, dstc1)
            pltpu.sync_copy(rows1, acc.at[dstc1], add=True)
            return carry2

        lax.fori_loop(0, IB // 2, body, 0)
        return carry

    lax.fori_loop(0, NB, outer, 0)
    plsc.subcore_barrier()
    pltpu.sync_copy(acc.at[pl.ds(s * ZROWS, ZROWS)], out_hbm.at[wid])


@functools.partial(
    pl.kernel, mesh=_MESH,
    out_type=jax.ShapeDtypeStruct((NW, ZROWS, D), jnp.float32),
    scratch_types=[
        pltpu.VMEM((K,), jnp.int32),
        pltpu.VMEM((K, D), jnp.float32),
        pltpu.VMEM_SHARED((NPAD, D), jnp.float32),
    ],
)
def _deg(ones_hbm, dst_hbm, zeros_hbm, out_hbm, dst_v, ones_v, acc):
    c = lax.axis_index("c")
    s = lax.axis_index("s")
    wid = c * NS + s
    pltpu.sync_copy(zeros_hbm.at[pl.ds(s * ZROWS, ZROWS)],
                    acc.at[pl.ds(s * ZROWS, ZROWS)])
    pltpu.sync_copy(ones_hbm, ones_v)
    plsc.subcore_barrier()

    def body(i, carry):
        base = wid * EPT + i * K
        pltpu.sync_copy(dst_hbm.at[pl.ds(base, K)], dst_v)
        pltpu.sync_copy(ones_v, acc.at[dst_v], add=True)
        return carry

    lax.fori_loop(0, CHUNKS, body, 0)
    plsc.subcore_barrier()
    pltpu.sync_copy(acc.at[pl.ds(s * ZROWS, ZROWS)], out_hbm.at[wid])


def _layer0_body(x_ref, a_ref, b_ref, da_ref, db_ref, ws_ref, wn_ref,
                 h_ref, inv_ref):
    inv = 1.0 / jnp.maximum(da_ref[...] + db_ref[...], 1.0)
    mean = (a_ref[...] + b_ref[...]) * inv
    h = (jnp.dot(x_ref[...], ws_ref[...], preferred_element_type=jnp.float32)
         + jnp.dot(mean, wn_ref[...], preferred_element_type=jnp.float32))
    h_ref[...] = jnp.maximum(h, 0.0)
    inv_ref[...] = inv


def _layerN_body(h_ref, a_ref, b_ref, inv_ref, ws_ref, wn_ref, o_ref, *, relu):
    mean = (a_ref[...] + b_ref[...]) * inv_ref[...]
    o = (jnp.dot(h_ref[...], ws_ref[...], preferred_element_type=jnp.float32)
         + jnp.dot(mean, wn_ref[...], preferred_element_type=jnp.float32))
    if relu:
        o = jnp.maximum(o, 0.0)
    o_ref[...] = o


_BLK = ZROWS
_GRID = NPAD // _BLK


def _row_spec(w):
    return pl.BlockSpec((_BLK, w), lambda i: (i, 0))


def _full_spec(r, w):
    return pl.BlockSpec((r, w), lambda i: (0, 0))


def _tc_layer0(x, accA, accB, degA, degB, ws, wn):
    return pl.pallas_call(
        _layer0_body,
        grid=(_GRID,),
        in_specs=[_row_spec(128), _row_spec(128), _row_spec(128),
                  _row_spec(128), _row_spec(128),
                  _full_spec(128, 128), _full_spec(128, 128)],
        out_specs=[_row_spec(128), _row_spec(128)],
        out_shape=[jax.ShapeDtypeStruct((NPAD, 128), jnp.float32),
                   jax.ShapeDtypeStruct((NPAD, 128), jnp.float32)],
    )(x, accA, accB, degA, degB, ws, wn)


def _tc_layerN(h, accA, accB, invb, ws, wn, relu):
    return pl.pallas_call(
        functools.partial(_layerN_body, relu=relu),
        grid=(_GRID,),
        in_specs=[_row_spec(128), _row_spec(128), _row_spec(128),
                  _row_spec(128), _full_spec(128, 128), _full_spec(128, 128)],
        out_specs=_row_spec(128),
        out_shape=jax.ShapeDtypeStruct((NPAD, 128), jnp.float32),
    )(h, accA, accB, invb, ws, wn)


def kernel(x, edge_index, W_self0, W_neigh0, W_self1, W_neigh1, W_self2, W_neigh2):
    src = edge_index[0]
    dst = edge_index[1]
    pad = E_PAD - E
    pad_src = (jnp.arange(pad, dtype=jnp.int32) * 97) % N
    pad_dst = N + (jnp.arange(pad, dtype=jnp.int32) % DUMP)
    srcp = jnp.concatenate([src, pad_src])
    dstp = jnp.concatenate([dst, pad_dst])
    zeros = jnp.zeros((NPAD, D), jnp.float32)
    ones = jnp.ones((K, D), jnp.float32)
    xp = jnp.pad(x, ((0, DUMP), (0, 0)))

    degr = _deg(ones, dstp, zeros).reshape(NC, NPAD, D)

    acc0 = _seg(xp, srcp, dstp, zeros).reshape(NC, NPAD, D)
    h1, invb = _tc_layer0(xp, acc0[0], acc0[1], degr[0], degr[1],
                          W_self0, W_neigh0)

    acc1 = _seg(h1, srcp, dstp, zeros).reshape(NC, NPAD, D)
    h2 = _tc_layerN(h1, acc1[0], acc1[1], invb, W_self1, W_neigh1, relu=True)

    acc2 = _seg(h2, srcp, dstp, zeros).reshape(NC, NPAD, D)
    ws2 = jnp.pad(W_self2, ((0, 0), (0, 128 - 40)))
    wn2 = jnp.pad(W_neigh2, ((0, 0), (0, 128 - 40)))
    h3 = _tc_layerN(h2, acc2[0], acc2[1], invb, ws2, wn2, relu=False)
    return h3[:N, :40]


# speedup vs baseline: 1.1262x; 1.0526x over previous
import functools

import jax
import jax.numpy as jnp
from jax import lax
from jax.experimental import pallas as pl
from jax.experimental.pallas import tpu as pltpu
from jax.experimental.pallas import tpu_sc as plsc

N = 10000
E = 320000
D = 128

NC = 2
NS = 16
NW = NC * NS
K = 128
H = K // 2
CHUNKS = 80
EPT = CHUNKS * K
E_PAD = NW * EPT
NPAD = 10112
ZROWS = NPAD // NS
DUMP = NPAD - N
IB = 16
NB = CHUNKS // IB

_MESH = plsc.VectorSubcoreMesh(core_axis_name="c", subcore_axis_name="s")


def _vcopy_idx(src_v, i, dst_v):
    base = pl.multiple_of(i * K, K)
    for k in range(K // 16):
        dst_v[pl.ds(k * 16, 16)] = src_v[pl.ds(base + k * 16, 16)]


@functools.partial(
    pl.kernel, mesh=_MESH,
    out_type=jax.ShapeDtypeStruct((NW, ZROWS, D), jnp.float32),
    scratch_types=[
        pltpu.VMEM((IB * K,), jnp.int32),
        pltpu.VMEM((IB * K,), jnp.int32),
        pltpu.VMEM((K,), jnp.int32),
        pltpu.VMEM((K,), jnp.int32),
        pltpu.VMEM((K, D), jnp.float32),
        pltpu.VMEM((K, D), jnp.float32),
        pltpu.VMEM_SHARED((NPAD, D), jnp.float32),
        pltpu.SemaphoreType.DMA,
        pltpu.SemaphoreType.DMA,
    ],
)
def _seg(table_hbm, src_hbm, dst_hbm, zeros_hbm, out_hbm,
         srcs_v, dsts_v, dstc0, dstc1, rows0, rows1, acc, sem0, sem1):
    c = lax.axis_index("c")
    s = lax.axis_index("s")
    wid = c * NS + s
    pltpu.sync_copy(zeros_hbm.at[pl.ds(s * ZROWS, ZROWS)],
                    acc.at[pl.ds(s * ZROWS, ZROWS)])
    plsc.subcore_barrier()

    def fire(l, rows, sem):
        base = pl.multiple_of(l * K, K)
        for q in range(2):
            pltpu.async_copy(table_hbm.at[srcs_v.at[pl.ds(base + q * H, H)]],
                             rows.at[pl.ds(q * H, H)], sem)

    def wait(rows, sem):
        pltpu.make_async_copy(table_hbm.at[srcs_v.at[pl.ds(0, K)]],
                              rows, sem).wait()

    def outer(b, carry):
        blk = wid * EPT + b * (IB * K)
        pltpu.sync_copy(src_hbm.at[pl.ds(blk, IB * K)], srcs_v)
        pltpu.sync_copy(dst_hbm.at[pl.ds(blk, IB * K)], dsts_v)
        fire(0, rows0, sem0)

        def body(i, carry2):
            l0 = 2 * i
            l1 = l0 + 1
            wait(rows0, sem0)
            fire(l1, rows1, sem1)
            _vcopy_idx(dsts_v, l0, dstc0)
            pltpu.sync_copy(rows0, acc.at[dstc0], add=True)
            wait(rows1, sem1)

            @pl.when(i < IB // 2 - 1)
            def _():
                fire(l0 + 2, rows0, sem0)

            _vcopy_idx(dsts_v, l1, dstc1)
            pltpu.sync_copy(rows1, acc.at[dstc1], add=True)
            return carry2

        lax.fori_loop(0, IB // 2, body, 0)
        return carry

    lax.fori_loop(0, NB, outer, 0)
    plsc.subcore_barrier()
    pltpu.sync_copy(acc.at[pl.ds(s * ZROWS, ZROWS)], out_hbm.at[wid])


@functools.partial(
    pl.kernel, mesh=_MESH,
    out_type=jax.ShapeDtypeStruct((NW, ZROWS, D), jnp.float32),
    scratch_types=[
        pltpu.VMEM((EPT,), jnp.int32),
        pltpu.VMEM((K,), jnp.int32),
        pltpu.VMEM((K, D), jnp.float32),
        pltpu.VMEM_SHARED((NPAD, D), jnp.float32),
    ],
)
def _deg(ones_hbm, dst_hbm, zeros_hbm, out_hbm, dsts_v, dstc, ones_v, acc):
    c = lax.axis_index("c")
    s = lax.axis_index("s")
    wid = c * NS + s
    pltpu.sync_copy(zeros_hbm.at[pl.ds(s * ZROWS, ZROWS)],
                    acc.at[pl.ds(s * ZROWS, ZROWS)])
    pltpu.sync_copy(dst_hbm.at[pl.ds(wid * EPT, EPT)], dsts_v)
    pltpu.sync_copy(ones_hbm, ones_v)
    plsc.subcore_barrier()

    def body(i, carry):
        _vcopy_idx(dsts_v, i, dstc)
        pltpu.sync_copy(ones_v, acc.at[dstc], add=True)
        return carry

    lax.fori_loop(0, CHUNKS, body, 0)
    plsc.subcore_barrier()
    pltpu.sync_copy(acc.at[pl.ds(s * ZROWS, ZROWS)], out_hbm.at[wid])


def _layer0_body(x_ref, a_ref, b_ref, da_ref, db_ref, ws_ref, wn_ref,
                 h_ref, inv_ref):
    inv = 1.0 / jnp.maximum(da_ref[...] + db_ref[...], 1.0)
    mean = (a_ref[...] + b_ref[...]) * inv
    h = (jnp.dot(x_ref[...], ws_ref[...], preferred_element_type=jnp.float32)
         + jnp.dot(mean, wn_ref[...], preferred_element_type=jnp.float32))
    h_ref[...] = jnp.maximum(h, 0.0)
    inv_ref[...] = inv


def _layerN_body(h_ref, a_ref, b_ref, inv_ref, ws_ref, wn_ref, o_ref, *, relu):
    mean = (a_ref[...] + b_ref[...]) * inv_ref[...]
    o = (jnp.dot(h_ref[...], ws_ref[...], preferred_element_type=jnp.float32)
         + jnp.dot(mean, wn_ref[...], preferred_element_type=jnp.float32))
    if relu:
        o = jnp.maximum(o, 0.0)
    o_ref[...] = o


_BLK = ZROWS
_GRID = NPAD // _BLK


def _row_spec(w):
    return pl.BlockSpec((_BLK, w), lambda i: (i, 0))


def _full_spec(r, w):
    return pl.BlockSpec((r, w), lambda i: (0, 0))


def _tc_layer0(x, accA, accB, degA, degB, ws, wn):
    return pl.pallas_call(
        _layer0_body,
        grid=(_GRID,),
        in_specs=[_row_spec(128), _row_spec(128), _row_spec(128),
                  _row_spec(128), _row_spec(128),
                  _full_spec(128, 128), _full_spec(128, 128)],
        out_specs=[_row_spec(128), _row_spec(128)],
        out_shape=[jax.ShapeDtypeStruct((NPAD, 128), jnp.float32),
                   jax.ShapeDtypeStruct((NPAD, 128), jnp.float32)],
    )(x, accA, accB, degA, degB, ws, wn)


def _tc_layerN(h, accA, accB, invb, ws, wn, relu):
    return pl.pallas_call(
        functools.partial(_layerN_body, relu=relu),
        grid=(_GRID,),
        in_specs=[_row_spec(128), _row_spec(128), _row_spec(128),
                  _row_spec(128), _full_spec(128, 128), _full_spec(128, 128)],
        out_specs=_row_spec(128),
        out_shape=jax.ShapeDtypeStruct((NPAD, 128), jnp.float32),
    )(h, accA, accB, invb, ws, wn)


def kernel(x, edge_index, W_self0, W_neigh0, W_self1, W_neigh1, W_self2, W_neigh2):
    src = edge_index[0]
    dst = edge_index[1]
    pad = E_PAD - E
    pad_src = (jnp.arange(pad, dtype=jnp.int32) * 97) % N
    pad_dst = N + (jnp.arange(pad, dtype=jnp.int32) % DUMP)
    srcp = jnp.concatenate([src, pad_src])
    dstp = jnp.concatenate([dst, pad_dst])
    zeros = jnp.zeros((NPAD, D), jnp.float32)
    ones = jnp.ones((K, D), jnp.float32)
    xp = jnp.pad(x, ((0, DUMP), (0, 0)))

    degr = _deg(ones, dstp, zeros).reshape(NC, NPAD, D)

    acc0 = _seg(xp, srcp, dstp, zeros).reshape(NC, NPAD, D)
    h1, invb = _tc_layer0(xp, acc0[0], acc0[1], degr[0], degr[1],
                          W_self0, W_neigh0)

    acc1 = _seg(h1, srcp, dstp, zeros).reshape(NC, NPAD, D)
    h2 = _tc_layerN(h1, acc1[0], acc1[1], invb, W_self1, W_neigh1, relu=True)

    acc2 = _seg(h2, srcp, dstp, zeros).reshape(NC, NPAD, D)
    ws2 = jnp.pad(W_self2, ((0, 0), (0, 128 - 40)))
    wn2 = jnp.pad(W_neigh2, ((0, 0), (0, 128 - 40)))
    h3 = _tc_layerN(h2, acc2[0], acc2[1], invb, ws2, wn2, relu=False)
    return h3[:N, :40]
